# Initial kernel scaffold; baseline (speedup 1.0000x reference)
import jax
import jax.numpy as jnp
from jax.experimental import pallas as pl


def kernel(x, Wg, W_gate, W1, W2):
    raise NotImplementedError("write your pallas kernel here")


# speedup vs baseline: 1.6120x; 1.6120x over previous
import jax
import jax.numpy as jnp
from jax.experimental import pallas as pl
from jax.experimental.pallas import tpu as pltpu

_NEG = -1e30


def _silu(v):
    return v * (1.0 / (1.0 + jnp.exp(-v)))


def _dense_body(x_ref, wg_ref, wgate_ref, w1_ref, w2t_ref, out_ref, tw_ref):
    e = pl.program_id(0)
    h = pl.program_id(1)
    n_exp = pl.num_programs(0)

    @pl.when((e == 0) & (h == 0))
    def _router():
        x = x_ref[...]
        logits = jnp.dot(x, wg_ref[...].T, preferred_element_type=jnp.float32)
        m = jnp.max(logits, axis=-1, keepdims=True)
        p = jnp.exp(logits - m)
        p = p / jnp.sum(p, axis=-1, keepdims=True)
        lane = jax.lax.broadcasted_iota(jnp.int32, p.shape, 1)
        p1 = jnp.max(p, axis=-1, keepdims=True)
        i1 = jnp.argmax(p, axis=-1)
        oh1 = (lane == i1[:, None]).astype(jnp.float32)
        pm = jnp.where(oh1 > 0, _NEG, p)
        p2 = jnp.max(pm, axis=-1, keepdims=True)
        i2 = jnp.argmax(pm, axis=-1)
        oh2 = (lane == i2[:, None]).astype(jnp.float32)
        denom = p1 + p2 + 1e-8
        tw_ref[...] = (oh1 * p1 + oh2 * p2) / denom
        out_ref[...] = jnp.zeros_like(out_ref)

    x = x_ref[...]
    wgate = wgate_ref[0]
    w1 = w1_ref[0]
    w2t = w2t_ref[0]
    g = _silu(jnp.dot(x, wgate.T, preferred_element_type=jnp.float32))
    u = jnp.dot(x, w1.T, preferred_element_type=jnp.float32)
    partial = jnp.dot(g * u, w2t, preferred_element_type=jnp.float32)
    lane = jax.lax.broadcasted_iota(jnp.int32, tw_ref.shape, 1)
    tw_col = jnp.sum(
        jnp.where(lane == e, tw_ref[...], 0.0), axis=-1, keepdims=True
    )
    out_ref[...] += tw_col * partial


def kernel(x, Wg, W_gate, W1, W2):
    t, dim = x.shape
    n_exp, hid, _ = W_gate.shape
    n_ht = 3
    ht = hid // n_ht
    assert hid % n_ht == 0

    return pl.pallas_call(
        _dense_body,
        grid=(n_exp, n_ht),
        in_specs=[
            pl.BlockSpec((t, dim), lambda e, h: (0, 0)),
            pl.BlockSpec((n_exp, dim), lambda e, h: (0, 0)),
            pl.BlockSpec((1, ht, dim), lambda e, h: (e, h, 0)),
            pl.BlockSpec((1, ht, dim), lambda e, h: (e, h, 0)),
            pl.BlockSpec((1, ht, dim), lambda e, h: (e, h, 0)),
        ],
        out_specs=pl.BlockSpec((t, dim), lambda e, h: (0, 0)),
        out_shape=jax.ShapeDtypeStruct((t, dim), jnp.float32),
        scratch_shapes=[pltpu.VMEM((t, n_exp), jnp.float32)],
        compiler_params=pltpu.CompilerParams(
            dimension_semantics=("arbitrary", "arbitrary"),
        ),
    )(x, Wg, W_gate, W1, W2.transpose(0, 2, 1))
